# baseline (speedup 1.0000x reference)
import jax
import jax.numpy as jnp
from jax.experimental import pallas as pl


def _copy_body(pe_ref, o_ref):
    o_ref[...] = pe_ref[...]


def kernel(idx, pe):
    T = idx.shape[1]
    out = pl.pallas_call(
        _copy_body,
        out_shape=jax.ShapeDtypeStruct((T, pe.shape[1]), pe.dtype),
        in_specs=[pl.BlockSpec((T, pe.shape[1]), lambda: (0, 0))],
        out_specs=pl.BlockSpec((T, pe.shape[1]), lambda: (0, 0)),
    )(pe)
    return out[None, :, :]
